# baseline (device time: 91638 ns/iter reference)
import jax
import jax.numpy as jnp
from jax import lax
from jax.experimental import pallas as pl
from jax.experimental.pallas import tpu as pltpu

N_DEV = 8
N_SUB = 4


def kernel(x, w_mat):
    m, k_loc = x.shape
    n = w_mat.shape[1]
    chunk = m // N_DEV
    half = n // 2
    subw = half // N_SUB
    n_streams = 2 * N_SUB

    def body(x_ref, w_ref, out_ref,
             send_buf, comm_buf, send_sems, recv_sems):
        my = lax.axis_index("i")
        left = (my - 1) % N_DEV
        right = (my + 1) % N_DEV

        barrier_sem = pltpu.get_barrier_semaphore()
        for nbr in [left, right]:
            pl.semaphore_signal(
                barrier_sem, inc=1,
                device_id=(nbr,), device_id_type=pl.DeviceIdType.MESH,
            )
        pl.semaphore_wait(barrier_sem, 2)

        def part(c, col0, ncol):
            return jnp.dot(
                x_ref[pl.ds(c * chunk, chunk), :], w_ref[:, col0:col0 + ncol],
                preferred_element_type=jnp.float32,
            )

        def make_rdma(k, s, target):
            return pltpu.make_async_remote_copy(
                src_ref=send_buf.at[k, s % 2],
                dst_ref=comm_buf.at[k, s],
                send_sem=send_sems.at[k, s % 2],
                recv_sem=recv_sems.at[k, s],
                device_id=(target,),
                device_id_type=pl.DeviceIdType.MESH,
            )

        rdmas = [[None] * (N_DEV - 1) for _ in range(n_streams)]

        def hop(s):
            pr = part((my - 1 - s) % N_DEV, 0, half)
            pl_ = part((my + 1 + s) % N_DEV, half, half)
            for j in range(N_SUB):
                for d, pmat, tgt in ((0, pr, right), (1, pl_, left)):
                    k = d * N_SUB + j
                    stripe = pmat[:, j * subw:(j + 1) * subw]
                    if s > 0:
                        rdmas[k][s - 1].wait_recv()
                        stripe = stripe + comm_buf[k, s - 1]
                    if s >= 2:
                        rdmas[k][s - 2].wait_send()
                    send_buf[k, s % 2] = stripe
                    rdmas[k][s] = make_rdma(k, s, tgt)
                    rdmas[k][s].start()

        for s in range(N_DEV - 1):
            hop(s)

        own = part(my, 0, n)
        for j in range(N_SUB):
            for d in (0, 1):
                k = d * N_SUB + j
                col0 = d * half + j * subw
                rdmas[k][N_DEV - 2].wait_recv()
                y = own[:, col0:col0 + subw] + comm_buf[k, N_DEV - 2]
                out_ref[:, col0:col0 + subw] = y * jax.nn.sigmoid(y)

        for s in (N_DEV - 3, N_DEV - 2):
            for k in range(n_streams):
                rdmas[k][s].wait_send()

        def _exit(second_barrier):
            for nbr in [left, right]:
                pl.semaphore_signal(
                    second_barrier, inc=1,
                    device_id=(nbr,), device_id_type=pl.DeviceIdType.MESH,
                )
            pl.semaphore_wait(second_barrier, 2)

        pl.run_scoped(_exit, second_barrier=pltpu.SemaphoreType.REGULAR)

    return pl.pallas_call(
        body,
        out_shape=jax.ShapeDtypeStruct((chunk, n), jnp.float32),
        in_specs=[
            pl.BlockSpec(memory_space=pltpu.VMEM),
            pl.BlockSpec(memory_space=pltpu.VMEM),
        ],
        out_specs=pl.BlockSpec(memory_space=pltpu.VMEM),
        scratch_shapes=[
            pltpu.VMEM((n_streams, 2, chunk, subw), jnp.float32),
            pltpu.VMEM((n_streams, N_DEV - 1, chunk, subw), jnp.float32),
            pltpu.SemaphoreType.DMA((n_streams, 2)),
            pltpu.SemaphoreType.DMA((n_streams, N_DEV - 1)),
        ],
        compiler_params=pltpu.CompilerParams(collective_id=0),
    )(x, w_mat)


# device time: 74487 ns/iter; 1.2303x vs baseline; 1.2303x over previous
import jax
import jax.numpy as jnp
from jax import lax
from jax.experimental import pallas as pl
from jax.experimental.pallas import tpu as pltpu

N_DEV = 8
ORDERS = (("z", "y", "x"), ("y", "x", "z"), ("x", "z", "y"))
WIDTHS = (640, 640, 768)
COL0S = (0, 640, 1280)


def kernel(x, w_mat):
    m, k_loc = x.shape
    n = w_mat.shape[1]
    chunk = m // N_DEV

    def body(x_ref, w_ref, out_ref, *scr):
        my = lax.axis_index("i")
        q = my % 4
        by = q // 2
        bx = (q % 2) ^ by
        bz = my // 4
        mybits = {"x": bx, "y": by, "z": bz}

        def pos_from(bits):
            return 4 * bits["z"] + 2 * bits["y"] + (bits["x"] ^ bits["y"])

        def partner(dim):
            b = dict(mybits)
            b[dim] = 1 - b[dim]
            return pos_from(b)

        def owner(t, a_bit, b_bit, c_bit):
            da, db, dc = ORDERS[t]
            return pos_from({da: a_bit, db: b_bit, dc: c_bit})

        def part(opos, c0, w):
            return jnp.dot(
                x_ref[pl.ds(opos * chunk, chunk), :], w_ref[:, c0:c0 + w],
                preferred_element_type=jnp.float32,
            )

        barrier_sem = pltpu.get_barrier_semaphore()
        for dim in ("x", "y", "z"):
            pl.semaphore_signal(
                barrier_sem, inc=1,
                device_id=(partner(dim),), device_id_type=pl.DeviceIdType.MESH,
            )
        pl.semaphore_wait(barrier_sem, 3)

        T = [scr[8 * t: 8 * t + 8] for t in range(3)]
        rdmas = {}

        def exchange(t, phase, src, dst, dim):
            s1, r1, s2, r2, s3, r3, ssem, rsem = T[t]
            rdma = pltpu.make_async_remote_copy(
                src_ref=src, dst_ref=dst,
                send_sem=ssem.at[phase - 1], recv_sem=rsem.at[phase - 1],
                device_id=(partner(dim),),
                device_id_type=pl.DeviceIdType.MESH,
            )
            rdma.start()
            rdmas[(t, phase)] = rdma

        for t in range(3):
            da, db, dc = ORDERS[t]
            c0, w = COL0S[t], WIDTHS[t]
            s1, r1 = T[t][0], T[t][1]
            for j in range(4):
                opos = owner(t, 1 - mybits[da], j // 2, j % 2)
                s1[j * chunk:(j + 1) * chunk, :] = part(opos, c0, w)
            exchange(t, 1, s1, r1, da)

        for t in range(3):
            da, db, dc = ORDERS[t]
            c0, w = COL0S[t], WIDTHS[t]
            s2 = T[t][2]
            for j in range(2):
                opos = owner(t, mybits[da], 1 - mybits[db], j)
                s2[j * chunk:(j + 1) * chunk, :] = part(opos, c0, w)

        for t in range(3):
            da, db, dc = ORDERS[t]
            c0, w = COL0S[t], WIDTHS[t]
            r1, s2, r2 = T[t][1], T[t][2], T[t][3]
            rdmas[(t, 1)].wait_recv()
            for j in range(2):
                blk = 2 * (1 - mybits[db]) + j
                s2[j * chunk:(j + 1) * chunk, :] = (
                    s2[j * chunk:(j + 1) * chunk, :]
                    + r1[pl.ds(blk * chunk, chunk), :]
                )
            exchange(t, 2, s2, r2, db)

        for t in range(3):
            da, db, dc = ORDERS[t]
            c0, w = COL0S[t], WIDTHS[t]
            s3 = T[t][4]
            opos = owner(t, mybits[da], mybits[db], 1 - mybits[dc])
            s3[:, :] = part(opos, c0, w)
            out_ref[:, c0:c0 + w] = part(my, c0, w)

        for t in range(3):
            da, db, dc = ORDERS[t]
            c0, w = COL0S[t], WIDTHS[t]
            r1, r2, s3, r3 = T[t][1], T[t][3], T[t][4], T[t][5]
            rdmas[(t, 2)].wait_recv()
            blk1 = 2 * mybits[db] + (1 - mybits[dc])
            blk2 = 1 - mybits[dc]
            s3[:, :] = (
                s3[:, :]
                + r1[pl.ds(blk1 * chunk, chunk), :]
                + r2[pl.ds(blk2 * chunk, chunk), :]
            )
            exchange(t, 3, s3, r3, dc)

        for t in range(3):
            da, db, dc = ORDERS[t]
            c0, w = COL0S[t], WIDTHS[t]
            r1, r2, r3 = T[t][1], T[t][3], T[t][5]
            rdmas[(t, 3)].wait_recv()
            blk1 = 2 * mybits[db] + mybits[dc]
            blk2 = mybits[dc]
            y = (
                out_ref[:, c0:c0 + w]
                + r1[pl.ds(blk1 * chunk, chunk), :]
                + r2[pl.ds(blk2 * chunk, chunk), :]
                + r3[:, :]
            )
            out_ref[:, c0:c0 + w] = y * jax.nn.sigmoid(y)

        for t in range(3):
            for phase in (1, 2, 3):
                rdmas[(t, phase)].wait_send()

    scratch_shapes = []
    for t in range(3):
        w = WIDTHS[t]
        scratch_shapes += [
            pltpu.VMEM((4 * chunk, w), jnp.float32),
            pltpu.VMEM((4 * chunk, w), jnp.float32),
            pltpu.VMEM((2 * chunk, w), jnp.float32),
            pltpu.VMEM((2 * chunk, w), jnp.float32),
            pltpu.VMEM((chunk, w), jnp.float32),
            pltpu.VMEM((chunk, w), jnp.float32),
            pltpu.SemaphoreType.DMA((3,)),
            pltpu.SemaphoreType.DMA((3,)),
        ]

    return pl.pallas_call(
        body,
        out_shape=jax.ShapeDtypeStruct((chunk, n), jnp.float32),
        in_specs=[
            pl.BlockSpec(memory_space=pltpu.VMEM),
            pl.BlockSpec(memory_space=pltpu.VMEM),
        ],
        out_specs=pl.BlockSpec(memory_space=pltpu.VMEM),
        scratch_shapes=scratch_shapes,
        compiler_params=pltpu.CompilerParams(collective_id=0),
    )(x, w_mat)


# device time: 71665 ns/iter; 1.2787x vs baseline; 1.0394x over previous
import jax
import jax.numpy as jnp
from jax import lax
from jax.experimental import pallas as pl
from jax.experimental.pallas import tpu as pltpu

N_DEV = 8
ORDERS = (("z", "y", "x"), ("y", "x", "z"), ("x", "z", "y"))
WIDTHS = (704, 640, 704)
COL0S = (0, 704, 1344)


def kernel(x, w_mat):
    m, k_loc = x.shape
    n = w_mat.shape[1]
    chunk = m // N_DEV

    def body(x_ref, w_ref, out_ref, *scr):
        my = lax.axis_index("i")
        q = my % 4
        by = q // 2
        bx = (q % 2) ^ by
        bz = my // 4
        mybits = {"x": bx, "y": by, "z": bz}

        def pos_from(bits):
            return 4 * bits["z"] + 2 * bits["y"] + (bits["x"] ^ bits["y"])

        def partner(dim):
            b = dict(mybits)
            b[dim] = 1 - b[dim]
            return pos_from(b)

        def part(opos, c0, w):
            return jnp.dot(
                x_ref[pl.ds(opos * chunk, chunk), :], w_ref[:, c0:c0 + w],
                preferred_element_type=jnp.float32,
            )

        def bits_t(t):
            da, db, dc = ORDERS[t]
            return mybits[da], mybits[db], mybits[dc]

        def owner(t, a_bit, b_bit, c_bit):
            da, db, dc = ORDERS[t]
            return pos_from({da: a_bit, db: b_bit, dc: c_bit})

        barrier_sem = pltpu.get_barrier_semaphore()
        for dim in ("x", "y", "z"):
            pl.semaphore_signal(
                barrier_sem, inc=1,
                device_id=(partner(dim),), device_id_type=pl.DeviceIdType.MESH,
            )
        pl.semaphore_wait(barrier_sem, 3)

        T = [scr[8 * t: 8 * t + 8] for t in range(3)]
        rdmas = {}

        def exchange(t, sem_idx, src, dst, dim):
            ssem, rsem = T[t][6], T[t][7]
            rdma = pltpu.make_async_remote_copy(
                src_ref=src, dst_ref=dst,
                send_sem=ssem.at[sem_idx], recv_sem=rsem.at[sem_idx],
                device_id=(partner(dim),),
                device_id_type=pl.DeviceIdType.MESH,
            )
            rdma.start()
            rdmas[(t, sem_idx)] = rdma

        for t in range(3):
            da, db, dc = ORDERS[t]
            ba, bb, bc = bits_t(t)
            c0, w = COL0S[t], WIDTHS[t]
            s1, r1 = T[t][0], T[t][1]
            for p in range(4):
                ob = (1 - bb) if p < 2 else bb
                oc = (1 - bc) if p % 2 == 0 else bc
                opos = owner(t, 1 - ba, ob, oc)
                s1[p * chunk:(p + 1) * chunk, :] = part(opos, c0, w)
                exchange(t, p, s1.at[pl.ds(p * chunk, chunk)],
                         r1.at[pl.ds(p * chunk, chunk)], da)

        for t in range(3):
            ba, bb, bc = bits_t(t)
            c0, w = COL0S[t], WIDTHS[t]
            s2 = T[t][2]
            for p in range(2):
                oc = (1 - bc) if p == 0 else bc
                opos = owner(t, ba, 1 - bb, oc)
                s2[p * chunk:(p + 1) * chunk, :] = part(opos, c0, w)

        for t in range(3):
            da, db, dc = ORDERS[t]
            s2, r2 = T[t][2], T[t][3]
            r1 = T[t][1]
            rdmas[(t, 0)].wait_recv()
            rdmas[(t, 1)].wait_recv()
            for p in range(2):
                s2[p * chunk:(p + 1) * chunk, :] = (
                    s2[p * chunk:(p + 1) * chunk, :]
                    + r1[p * chunk:(p + 1) * chunk, :]
                )
                exchange(t, 4 + p, s2.at[pl.ds(p * chunk, chunk)],
                         r2.at[pl.ds(p * chunk, chunk)], db)

        for t in range(3):
            ba, bb, bc = bits_t(t)
            c0, w = COL0S[t], WIDTHS[t]
            s3 = T[t][4]
            s3[:, :] = part(owner(t, ba, bb, 1 - bc), c0, w)
            out_ref[:, c0:c0 + w] = part(my, c0, w)

        for t in range(3):
            da, db, dc = ORDERS[t]
            r1, r2, s3, r3 = T[t][1], T[t][3], T[t][4], T[t][5]
            rdmas[(t, 2)].wait_recv()
            rdmas[(t, 4)].wait_recv()
            s3[:, :] = (
                s3[:, :]
                + r1[2 * chunk:3 * chunk, :]
                + r2[0 * chunk:1 * chunk, :]
            )
            exchange(t, 6, s3, r3, dc)

        for t in range(3):
            c0, w = COL0S[t], WIDTHS[t]
            r1, r2, r3 = T[t][1], T[t][3], T[t][5]
            rdmas[(t, 3)].wait_recv()
            rdmas[(t, 5)].wait_recv()
            rdmas[(t, 6)].wait_recv()
            y = (
                out_ref[:, c0:c0 + w]
                + r1[3 * chunk:4 * chunk, :]
                + r2[1 * chunk:2 * chunk, :]
                + r3[:, :]
            )
            out_ref[:, c0:c0 + w] = y * jax.nn.sigmoid(y)

        for t in range(3):
            for idx in range(7):
                rdmas[(t, idx)].wait_send()

    scratch_shapes = []
    for t in range(3):
        w = WIDTHS[t]
        scratch_shapes += [
            pltpu.VMEM((4 * chunk, w), jnp.float32),
            pltpu.VMEM((4 * chunk, w), jnp.float32),
            pltpu.VMEM((2 * chunk, w), jnp.float32),
            pltpu.VMEM((2 * chunk, w), jnp.float32),
            pltpu.VMEM((chunk, w), jnp.float32),
            pltpu.VMEM((chunk, w), jnp.float32),
            pltpu.SemaphoreType.DMA((7,)),
            pltpu.SemaphoreType.DMA((7,)),
        ]

    return pl.pallas_call(
        body,
        out_shape=jax.ShapeDtypeStruct((chunk, n), jnp.float32),
        in_specs=[
            pl.BlockSpec(memory_space=pltpu.VMEM),
            pl.BlockSpec(memory_space=pltpu.VMEM),
        ],
        out_specs=pl.BlockSpec(memory_space=pltpu.VMEM),
        scratch_shapes=scratch_shapes,
        compiler_params=pltpu.CompilerParams(collective_id=0),
    )(x, w_mat)


# device time: 40610 ns/iter; 2.2565x vs baseline; 1.7647x over previous
import jax
import jax.numpy as jnp
from jax import lax
from jax.experimental import pallas as pl
from jax.experimental.pallas import tpu as pltpu

N_DEV = 8
ORDERS = (("z", "y", "x"), ("y", "x", "z"), ("x", "z", "y"))
WIDTHS = (704, 640, 704)
COL0S = (0, 704, 1344)


def kernel(x, w_mat):
    m, k_loc = x.shape
    n = w_mat.shape[1]
    chunk = m // N_DEV

    def body(x_ref, w_ref, out_ref, *scr):
        my = lax.axis_index("i")
        q = my % 4
        by = q // 2
        bx = (q % 2) ^ by
        bz = my // 4
        mybits = {"x": bx, "y": by, "z": bz}

        def pos_from(bits):
            return 4 * bits["z"] + 2 * bits["y"] + (bits["x"] ^ bits["y"])

        def partner(dim):
            b = dict(mybits)
            b[dim] = 1 - b[dim]
            return pos_from(b)

        def part(opos, c0, w):
            return jnp.dot(
                x_ref[pl.ds(opos * chunk, chunk), :], w_ref[:, c0:c0 + w],
                preferred_element_type=jnp.float32,
            )

        def bits_t(t):
            da, db, dc = ORDERS[t]
            return mybits[da], mybits[db], mybits[dc]

        def owner(t, a_bit, b_bit, c_bit):
            da, db, dc = ORDERS[t]
            return pos_from({da: a_bit, db: b_bit, dc: c_bit})

        barrier_sem = pltpu.get_barrier_semaphore()
        for dim in ("x", "y", "z"):
            pl.semaphore_signal(
                barrier_sem, inc=1,
                device_id=(partner(dim),), device_id_type=pl.DeviceIdType.MESH,
            )
        pl.semaphore_wait(barrier_sem, 3)

        T = [scr[8 * t: 8 * t + 8] for t in range(3)]
        rdmas = {}

        def exchange(t, sem_idx, src, dst, dim):
            ssem, rsem = T[t][6], T[t][7]
            rdma = pltpu.make_async_remote_copy(
                src_ref=src, dst_ref=dst,
                send_sem=ssem.at[sem_idx], recv_sem=rsem.at[sem_idx],
                device_id=(partner(dim),),
                device_id_type=pl.DeviceIdType.MESH,
            )
            rdma.start()
            rdmas[(t, sem_idx)] = rdma

        for p in range(4):
            for t in range(3):
                da, db, dc = ORDERS[t]
                ba, bb, bc = bits_t(t)
                c0, w = COL0S[t], WIDTHS[t]
                s1, r1 = T[t][0], T[t][1]
                ob = (1 - bb) if p < 2 else bb
                oc = (1 - bc) if p % 2 == 0 else bc
                opos = owner(t, 1 - ba, ob, oc)
                s1[p * chunk:(p + 1) * chunk, :] = part(opos, c0, w).astype(
                    jnp.bfloat16)
                exchange(t, p, s1.at[pl.ds(p * chunk, chunk)],
                         r1.at[pl.ds(p * chunk, chunk)], da)

        for t in range(3):
            ba, bb, bc = bits_t(t)
            c0, w = COL0S[t], WIDTHS[t]
            s2 = T[t][2]
            for p in range(2):
                oc = (1 - bc) if p == 0 else bc
                opos = owner(t, ba, 1 - bb, oc)
                s2[p * chunk:(p + 1) * chunk, :] = part(opos, c0, w).astype(
                    jnp.bfloat16)

        for t in range(3):
            da, db, dc = ORDERS[t]
            s2, r2 = T[t][2], T[t][3]
            r1 = T[t][1]
            rdmas[(t, 0)].wait_recv()
            rdmas[(t, 1)].wait_recv()
            for p in range(2):
                s2[p * chunk:(p + 1) * chunk, :] = (
                    s2[p * chunk:(p + 1) * chunk, :].astype(jnp.float32)
                    + r1[p * chunk:(p + 1) * chunk, :].astype(jnp.float32)
                ).astype(jnp.bfloat16)
                exchange(t, 4 + p, s2.at[pl.ds(p * chunk, chunk)],
                         r2.at[pl.ds(p * chunk, chunk)], db)

        for t in range(3):
            ba, bb, bc = bits_t(t)
            c0, w = COL0S[t], WIDTHS[t]
            s3 = T[t][4]
            s3[:, :] = part(owner(t, ba, bb, 1 - bc), c0, w).astype(jnp.bfloat16)
            out_ref[:, c0:c0 + w] = part(my, c0, w)

        for t in range(3):
            da, db, dc = ORDERS[t]
            r1, r2, s3, r3 = T[t][1], T[t][3], T[t][4], T[t][5]
            rdmas[(t, 2)].wait_recv()
            rdmas[(t, 4)].wait_recv()
            s3[:, :] = (
                s3[:, :].astype(jnp.float32)
                + r1[2 * chunk:3 * chunk, :].astype(jnp.float32)
                + r2[0 * chunk:1 * chunk, :].astype(jnp.float32)
            ).astype(jnp.bfloat16)
            exchange(t, 6, s3, r3, dc)

        for t in range(3):
            c0, w = COL0S[t], WIDTHS[t]
            r1, r2, r3 = T[t][1], T[t][3], T[t][5]
            rdmas[(t, 3)].wait_recv()
            rdmas[(t, 5)].wait_recv()
            rdmas[(t, 6)].wait_recv()
            y = (
                out_ref[:, c0:c0 + w]
                + r1[3 * chunk:4 * chunk, :].astype(jnp.float32)
                + r2[1 * chunk:2 * chunk, :].astype(jnp.float32)
                + r3[:, :].astype(jnp.float32)
            )
            out_ref[:, c0:c0 + w] = y * jax.nn.sigmoid(y)

        for t in range(3):
            for idx in range(7):
                rdmas[(t, idx)].wait_send()

    scratch_shapes = []
    for t in range(3):
        w = WIDTHS[t]
        scratch_shapes += [
            pltpu.VMEM((4 * chunk, w), jnp.bfloat16),
            pltpu.VMEM((4 * chunk, w), jnp.bfloat16),
            pltpu.VMEM((2 * chunk, w), jnp.bfloat16),
            pltpu.VMEM((2 * chunk, w), jnp.bfloat16),
            pltpu.VMEM((chunk, w), jnp.bfloat16),
            pltpu.VMEM((chunk, w), jnp.bfloat16),
            pltpu.SemaphoreType.DMA((7,)),
            pltpu.SemaphoreType.DMA((7,)),
        ]

    return pl.pallas_call(
        body,
        out_shape=jax.ShapeDtypeStruct((chunk, n), jnp.float32),
        in_specs=[
            pl.BlockSpec(memory_space=pltpu.VMEM),
            pl.BlockSpec(memory_space=pltpu.VMEM),
        ],
        out_specs=pl.BlockSpec(memory_space=pltpu.VMEM),
        scratch_shapes=scratch_shapes,
        compiler_params=pltpu.CompilerParams(collective_id=0),
    )(x, w_mat)
